# baseline (speedup 1.0000x reference)
import jax
import jax.numpy as jnp
from jax import lax
from jax.experimental import pallas as pl
from jax.experimental.pallas import tpu as pltpu
from jax.experimental.pallas import tpu_sc as plsc

EMB = 64
SCALE = 8.0
B, L = 4096, 200
VOCAB = 100000
NC, NS = 2, 16
NW = NC * NS
BC = B // NW
LB = 4
NG = L // LB


def _emb_body(table, toks, mid, idx_v, rows_v, si0, si1, sg0, sg1, so0, so1):
    wid = lax.axis_index("s") * NC + lax.axis_index("c")
    b0 = wid * BC
    s_idx = (si0, si1)
    s_g = (sg0, sg1)
    s_o = (so0, so1)

    def fire_idx(g, s):
        pltpu.async_copy(toks.at[pl.ds(g * LB, LB), pl.ds(b0, BC)],
                         idx_v.at[pl.ds(s * LB, LB)], s_idx[s])

    def wait_idx(g, s):
        pltpu.make_async_copy(toks.at[pl.ds(g * LB, LB), pl.ds(b0, BC)],
                              idx_v.at[pl.ds(s * LB, LB)], s_idx[s]).wait()

    def run_gathers(s):
        descs = [
            pltpu.async_copy(table.at[idx_v.at[s * LB + j]],
                             rows_v.at[s * LB + j], s_g[s])
            for j in range(LB)
        ]
        for d in descs:
            d.wait()

    def fire_out(g, s):
        pltpu.async_copy(rows_v.at[pl.ds(s * LB, LB)],
                         mid.at[pl.ds(g * LB, LB), pl.ds(b0, BC), pl.ds(0, EMB)],
                         s_o[s])

    def wait_out(g, s):
        pltpu.make_async_copy(rows_v.at[pl.ds(s * LB, LB)],
                              mid.at[pl.ds(g * LB, LB), pl.ds(b0, BC), pl.ds(0, EMB)],
                              s_o[s]).wait()

    fire_idx(0, 0)
    fire_idx(1, 1)
    for g in (0, 1):
        s = g
        wait_idx(g, s)
        run_gathers(s)
        fire_out(g, s)
        fire_idx(g + 2, s)

    @pl.loop(0, (NG - 4) // 2)
    def _steady(i):
        for s in range(2):
            g = 2 + i * 2 + s
            wait_idx(g, s)
            wait_out(g - 2, s)
            run_gathers(s)
            fire_out(g, s)
            fire_idx(g + 2, s)

    for g in (NG - 2, NG - 1):
        s = g % 2
        wait_idx(g, s)
        wait_out(g - 2, s)
        run_gathers(s)
        fire_out(g, s)
    for g in (NG - 2, NG - 1):
        wait_out(g, g % 2)


def _emb_gather(table, toks_t):
    mesh = plsc.VectorSubcoreMesh(core_axis_name="c", subcore_axis_name="s",
                                  num_cores=NC, num_subcores=NS)
    f = pl.kernel(
        _emb_body,
        out_type=jax.ShapeDtypeStruct((L, B, 2 * EMB), jnp.float32),
        mesh=mesh,
        scratch_types=[
            pltpu.VMEM((2 * LB, BC), jnp.int32),
            pltpu.VMEM((2 * LB, 128, EMB), jnp.float32),
            pltpu.SemaphoreType.DMA,
            pltpu.SemaphoreType.DMA,
            pltpu.SemaphoreType.DMA,
            pltpu.SemaphoreType.DMA,
            pltpu.SemaphoreType.DMA,
            pltpu.SemaphoreType.DMA,
        ],
        compiler_params=pltpu.CompilerParams(use_tc_tiling_on_sc=False),
    )
    return f(table, toks_t)


def _tr_body(x_ref, o_ref):
    r = lax.broadcasted_iota(jnp.int32, (EMB, EMB), 0)
    c = lax.broadcasted_iota(jnp.int32, (EMB, EMB), 1)
    eye = jnp.where(r == c, SCALE, 0.0).astype(jnp.float32)
    x = x_ref[0][:, :EMB]
    o_ref[0] = lax.dot_general(eye, x, (((1,), (1,)), ((), ())),
                               precision=lax.Precision.HIGHEST,
                               preferred_element_type=jnp.float32)


def _transpose_scale(mid):
    return pl.pallas_call(
        _tr_body,
        out_shape=jax.ShapeDtypeStruct((L, EMB, B), jnp.float32),
        grid=(L,),
        in_specs=[pl.BlockSpec((1, B, 2 * EMB), lambda i: (i, 0, 0))],
        out_specs=pl.BlockSpec((1, EMB, B), lambda i: (i, 0, 0)),
    )(mid)


def kernel(tokens, table):
    toks_t = jnp.transpose(tokens)
    mid = _emb_gather(table, toks_t)
    out_t = _transpose_scale(mid)
    return jnp.transpose(out_t, (2, 0, 1))


# speedup vs baseline: 1.3702x; 1.3702x over previous
import jax
import jax.numpy as jnp
from jax import lax
from jax.experimental import pallas as pl
from jax.experimental.pallas import tpu as pltpu
from jax.experimental.pallas import tpu_sc as plsc

EMB = 64
SCALE = 8.0
B, L = 4096, 200
VOCAB = 100000
NC, NS = 2, 16
NW = NC * NS
BC = B // NW
LB = 4
NG = L // LB


def _emb_body(table, toks, mid, idx_v, rows_v, si0, si1, sg0, sg1, so0, so1):
    wid = lax.axis_index("s") * NC + lax.axis_index("c")
    b0 = wid * BC
    s_idx = (si0, si1)
    s_g = (sg0, sg1)
    s_o = (so0, so1)

    def fire_idx(g, s):
        pltpu.async_copy(toks.at[pl.ds(g * LB, LB), pl.ds(b0, BC)],
                         idx_v.at[pl.ds(s * LB, LB)], s_idx[s])

    def wait_idx(g, s):
        pltpu.make_async_copy(toks.at[pl.ds(g * LB, LB), pl.ds(b0, BC)],
                              idx_v.at[pl.ds(s * LB, LB)], s_idx[s]).wait()

    def run_gathers(s):
        descs = [
            pltpu.async_copy(table.at[idx_v.at[s * LB + j]],
                             rows_v.at[s * LB + j], s_g[s])
            for j in range(LB)
        ]
        for d in descs:
            d.wait()

    def fire_out(g, s):
        pltpu.async_copy(rows_v.at[pl.ds(s * LB, LB)],
                         mid.at[pl.ds(g * LB, LB), pl.ds(b0, BC), pl.ds(0, EMB)],
                         s_o[s])

    def wait_out(g, s):
        pltpu.make_async_copy(rows_v.at[pl.ds(s * LB, LB)],
                              mid.at[pl.ds(g * LB, LB), pl.ds(b0, BC), pl.ds(0, EMB)],
                              s_o[s]).wait()

    fire_idx(0, 0)
    fire_idx(1, 1)
    for g in (0, 1):
        s = g
        wait_idx(g, s)
        run_gathers(s)
        fire_out(g, s)
        fire_idx(g + 2, s)

    @pl.loop(0, (NG - 4) // 2)
    def _steady(i):
        for s in range(2):
            g = 2 + i * 2 + s
            wait_idx(g, s)
            wait_out(g - 2, s)
            run_gathers(s)
            fire_out(g, s)
            fire_idx(g + 2, s)

    for g in (NG - 2, NG - 1):
        s = g % 2
        wait_idx(g, s)
        wait_out(g - 2, s)
        run_gathers(s)
        fire_out(g, s)
    for g in (NG - 2, NG - 1):
        wait_out(g, g % 2)


def _emb_gather(table, toks_t):
    mesh = plsc.VectorSubcoreMesh(core_axis_name="c", subcore_axis_name="s",
                                  num_cores=NC, num_subcores=NS)
    f = pl.kernel(
        _emb_body,
        out_type=jax.ShapeDtypeStruct((L, B, 2 * EMB), jnp.float32),
        mesh=mesh,
        scratch_types=[
            pltpu.VMEM((2 * LB, BC), jnp.int32),
            pltpu.VMEM((2 * LB, 128, EMB), jnp.float32),
            pltpu.SemaphoreType.DMA,
            pltpu.SemaphoreType.DMA,
            pltpu.SemaphoreType.DMA,
            pltpu.SemaphoreType.DMA,
            pltpu.SemaphoreType.DMA,
            pltpu.SemaphoreType.DMA,
        ],
        compiler_params=pltpu.CompilerParams(use_tc_tiling_on_sc=False),
    )
    return f(table, toks_t)


def _tr_body(x_ref, o_ref):
    x = x_ref[0][:, :EMB]
    o_ref[0] = jnp.transpose(x) * SCALE


def _transpose_scale(mid):
    return pl.pallas_call(
        _tr_body,
        out_shape=jax.ShapeDtypeStruct((L, EMB, B), jnp.float32),
        grid=(L,),
        in_specs=[pl.BlockSpec((1, B, 2 * EMB), lambda i: (i, 0, 0))],
        out_specs=pl.BlockSpec((1, EMB, B), lambda i: (i, 0, 0)),
    )(mid)


def kernel(tokens, table):
    toks_t = jnp.transpose(tokens)
    mid = _emb_gather(table, toks_t)
    out_t = _transpose_scale(mid)
    return jnp.transpose(out_t, (2, 0, 1))


# speedup vs baseline: 1.5519x; 1.1326x over previous
import jax
import jax.numpy as jnp
from jax import lax
from jax.experimental import pallas as pl
from jax.experimental.pallas import tpu as pltpu
from jax.experimental.pallas import tpu_sc as plsc

EMB = 64
SCALE = 8.0
B, L = 4096, 200
VOCAB = 100000
NC, NS = 2, 16
NW = NC * NS
BC = B // NW
LB = 4
NG = L // LB


def _emb_body(table, toks, mid, idx_v, rows_v, si0, si1, sg0, sg1, so0, so1):
    wid = lax.axis_index("s") * NC + lax.axis_index("c")
    b0 = wid * BC
    s_idx = (si0, si1)
    s_g = (sg0, sg1)
    s_o = (so0, so1)

    def fire_idx(g, s):
        pltpu.async_copy(toks.at[pl.ds(g * LB, LB), pl.ds(b0, BC)],
                         idx_v.at[pl.ds(s * LB, LB)], s_idx[s])

    def wait_idx(g, s):
        pltpu.make_async_copy(toks.at[pl.ds(g * LB, LB), pl.ds(b0, BC)],
                              idx_v.at[pl.ds(s * LB, LB)], s_idx[s]).wait()

    def run_gathers(s):
        descs = [
            pltpu.async_copy(table.at[idx_v.at[s * LB + j]],
                             rows_v.at[s * LB + j], s_g[s])
            for j in range(LB)
        ]
        for d in descs:
            d.wait()

    def fire_out(g, s):
        pltpu.async_copy(rows_v.at[pl.ds(s * LB, LB)],
                         mid.at[pl.ds(g * LB, LB), pl.ds(b0, BC), pl.ds(0, EMB)],
                         s_o[s])

    def wait_out(g, s):
        pltpu.make_async_copy(rows_v.at[pl.ds(s * LB, LB)],
                              mid.at[pl.ds(g * LB, LB), pl.ds(b0, BC), pl.ds(0, EMB)],
                              s_o[s]).wait()

    fire_idx(0, 0)
    fire_idx(1, 1)
    for g in (0, 1):
        s = g
        wait_idx(g, s)
        run_gathers(s)
        fire_out(g, s)
        fire_idx(g + 2, s)

    @pl.loop(0, (NG - 4) // 2)
    def _steady(i):
        for s in range(2):
            g = 2 + i * 2 + s
            wait_idx(g, s)
            wait_out(g - 2, s)
            run_gathers(s)
            fire_out(g, s)
            fire_idx(g + 2, s)

    for g in (NG - 2, NG - 1):
        s = g % 2
        wait_idx(g, s)
        wait_out(g - 2, s)
        run_gathers(s)
        fire_out(g, s)
    for g in (NG - 2, NG - 1):
        wait_out(g, g % 2)


def _emb_gather(table, toks_t):
    mesh = plsc.VectorSubcoreMesh(core_axis_name="c", subcore_axis_name="s",
                                  num_cores=NC, num_subcores=NS)
    f = pl.kernel(
        _emb_body,
        out_type=jax.ShapeDtypeStruct((L, B, 2 * EMB), jnp.float32),
        mesh=mesh,
        scratch_types=[
            pltpu.VMEM((2 * LB, BC), jnp.int32),
            pltpu.VMEM((2 * LB, 128, EMB), jnp.float32),
            pltpu.SemaphoreType.DMA,
            pltpu.SemaphoreType.DMA,
            pltpu.SemaphoreType.DMA,
            pltpu.SemaphoreType.DMA,
            pltpu.SemaphoreType.DMA,
            pltpu.SemaphoreType.DMA,
        ],
        compiler_params=pltpu.CompilerParams(use_tc_tiling_on_sc=False),
    )
    return f(table, toks_t)


def _tr_body(x_ref, o_ref):
    for j in range(_TL):
        x = x_ref[j][:, :EMB]
        o_ref[j] = jnp.transpose(x) * SCALE


_TL = 2


def _transpose_scale(mid):
    return pl.pallas_call(
        _tr_body,
        out_shape=jax.ShapeDtypeStruct((L, EMB, B), jnp.float32),
        grid=(L // _TL,),
        in_specs=[pl.BlockSpec((_TL, B, 2 * EMB), lambda i: (i, 0, 0))],
        out_specs=pl.BlockSpec((_TL, EMB, B), lambda i: (i, 0, 0)),
    )(mid)


def kernel(tokens, table):
    toks_t = jnp.transpose(tokens)
    mid = _emb_gather(table, toks_t)
    out_t = _transpose_scale(mid)
    return jnp.transpose(out_t, (2, 0, 1))


# speedup vs baseline: 1.5818x; 1.0193x over previous
import jax
import jax.numpy as jnp
from jax import lax
from jax.experimental import pallas as pl
from jax.experimental.pallas import tpu as pltpu
from jax.experimental.pallas import tpu_sc as plsc

EMB = 64
SCALE = 8.0
B, L = 4096, 200
VOCAB = 100000
NC, NS = 2, 16
NW = NC * NS
BC = B // NW
LB = 4
NG = L // LB


def _emb_body(table, toks, mid, idx_v, rows_v, si0, si1, sg0, sg1, so0, so1):
    wid = lax.axis_index("s") * NC + lax.axis_index("c")
    b0 = wid * BC
    s_idx = (si0, si1)
    s_g = (sg0, sg1)
    s_o = (so0, so1)

    def fire_idx(g, s):
        pltpu.async_copy(toks.at[pl.ds(g * LB, LB), pl.ds(b0, BC)],
                         idx_v.at[pl.ds(s * LB, LB)], s_idx[s])

    def wait_idx(g, s):
        pltpu.make_async_copy(toks.at[pl.ds(g * LB, LB), pl.ds(b0, BC)],
                              idx_v.at[pl.ds(s * LB, LB)], s_idx[s]).wait()

    def run_gathers(s):
        descs = [
            pltpu.async_copy(table.at[idx_v.at[s * LB + j]],
                             rows_v.at[s * LB + j], s_g[s])
            for j in range(LB)
        ]
        for d in descs:
            d.wait()

    def fire_out(g, s):
        pltpu.async_copy(rows_v.at[pl.ds(s * LB, LB)],
                         mid.at[pl.ds(g * LB, LB), pl.ds(b0, BC), pl.ds(0, EMB)],
                         s_o[s])

    def wait_out(g, s):
        pltpu.make_async_copy(rows_v.at[pl.ds(s * LB, LB)],
                              mid.at[pl.ds(g * LB, LB), pl.ds(b0, BC), pl.ds(0, EMB)],
                              s_o[s]).wait()

    fire_idx(0, 0)
    fire_idx(1, 1)
    for g in (0, 1):
        s = g
        wait_idx(g, s)
        run_gathers(s)
        fire_out(g, s)
        fire_idx(g + 2, s)

    @pl.loop(0, (NG - 4) // 2)
    def _steady(i):
        for s in range(2):
            g = 2 + i * 2 + s
            wait_idx(g, s)
            wait_out(g - 2, s)
            run_gathers(s)
            fire_out(g, s)
            fire_idx(g + 2, s)

    for g in (NG - 2, NG - 1):
        s = g % 2
        wait_idx(g, s)
        wait_out(g - 2, s)
        run_gathers(s)
        fire_out(g, s)
    for g in (NG - 2, NG - 1):
        wait_out(g, g % 2)


def _emb_gather(table, toks_t):
    mesh = plsc.VectorSubcoreMesh(core_axis_name="c", subcore_axis_name="s",
                                  num_cores=NC, num_subcores=NS)
    f = pl.kernel(
        _emb_body,
        out_type=jax.ShapeDtypeStruct((L, B, 2 * EMB), jnp.float32),
        mesh=mesh,
        scratch_types=[
            pltpu.VMEM((2 * LB, BC), jnp.int32),
            pltpu.VMEM((2 * LB, 128, EMB), jnp.float32),
            pltpu.SemaphoreType.DMA,
            pltpu.SemaphoreType.DMA,
            pltpu.SemaphoreType.DMA,
            pltpu.SemaphoreType.DMA,
            pltpu.SemaphoreType.DMA,
            pltpu.SemaphoreType.DMA,
        ],
        compiler_params=pltpu.CompilerParams(use_tc_tiling_on_sc=False),
    )
    return f(table, toks_t)


def _tr_body(x_ref, o_ref):
    for j in range(_TL):
        x = x_ref[j][:, :EMB]
        o_ref[j] = jnp.transpose(x) * SCALE


_TL = 4


def _transpose_scale(mid):
    return pl.pallas_call(
        _tr_body,
        out_shape=jax.ShapeDtypeStruct((L, EMB, B), jnp.float32),
        grid=(L // _TL,),
        in_specs=[pl.BlockSpec((_TL, B, 2 * EMB), lambda i: (i, 0, 0))],
        out_specs=pl.BlockSpec((_TL, EMB, B), lambda i: (i, 0, 0)),
    )(mid)


def kernel(tokens, table):
    toks_t = jnp.transpose(tokens)
    mid = _emb_gather(table, toks_t)
    out_t = _transpose_scale(mid)
    return jnp.transpose(out_t, (2, 0, 1))
